# baseline (speedup 1.0000x reference)
import jax
import jax.numpy as jnp
from jax import lax
from jax.experimental import pallas as pl
from jax.experimental.pallas import tpu as pltpu
from jax.experimental.pallas import tpu_sc as plsc

N = 5000
NMS_THRESH = 0.5
SCORE_THRESH = 0.05
DETS_PER_IMG = 100

L = 16
KMAX = 5008
BOXPAD = 4 * N + L
SPAD = 5152
OUTB_PAD = 512
OUTS_PAD = 128

B_PER_BLK = 5
NBLK = 64

_GDN = lax.GatherDimensionNumbers(offset_dims=(), collapsed_slice_dims=(0,),
                                  start_index_map=(0,))


def _vbcast(v, k):
    idx = jnp.full((L, 1), k, jnp.int32)
    return lax.gather(v, idx, _GDN, slice_sizes=(1,),
                      mode=lax.GatherScatterMode.PROMISE_IN_BOUNDS)


def _nms_body(boxes_hbm, scores_hbm, order_hbm, outb_hbm, outs_hbm,
              boxes_v, scores_v, order_v,
              kx0, ky0, kx1, ky1, karea,
              outb_v, outs_v, kcnt_s, ocnt_s, sem_b, sem_s, sem_o):
    cid = lax.axis_index("c")
    sid = lax.axis_index("s")

    @pl.when(jnp.logical_and(cid == 0, sid == 0))
    def _():
        cp_b = pltpu.async_copy(boxes_hbm, boxes_v.at[pl.ds(0, 4 * N)], sem_b)
        cp_s = pltpu.async_copy(scores_hbm, scores_v.at[pl.ds(0, N)], sem_s)
        cp_o = pltpu.async_copy(order_hbm, order_v.at[pl.ds(0, N)], sem_o)

        zv = jnp.zeros((L,), jnp.float32)
        lane = lax.iota(jnp.int32, L)
        lane0 = lane == 0
        for j in range(OUTB_PAD // L):
            outb_v[pl.ds(j * L, L)] = zv
        for j in range(OUTS_PAD // L):
            outs_v[pl.ds(j * L, L)] = zv
        kcnt_s[0] = jnp.int32(0)
        ocnt_s[0] = jnp.int32(0)
        cp_b.wait()
        cp_s.wait()
        cp_o.wait()

        for j in range((SPAD - N + L - 1) // L - 1):
            order_v[pl.ds(N + j * L, L)] = jnp.zeros((L,), jnp.int32)

        def batch(b, bi):
            b16 = (bi * B_PER_BLK + b) * L
            oiv = order_v[pl.ds(b16, L)]
            ob = oiv * 4
            gx0 = plsc.load_gather(boxes_v, [ob])
            gy0 = plsc.load_gather(boxes_v, [ob + 1])
            gx1 = plsc.load_gather(boxes_v, [ob + 2])
            gy1 = plsc.load_gather(boxes_v, [ob + 3])
            gs = plsc.load_gather(scores_v, [oiv])

            for l in range(L):
                kc = kcnt_s[0]
                co = ocnt_s[0]
                act = jnp.logical_and(co < DETS_PER_IMG, b16 + l < N)

                bx0 = _vbcast(gx0, l)
                by0 = _vbcast(gy0, l)
                bx1 = _vbcast(gx1, l)
                by1 = _vbcast(gy1, l)
                barea = (bx1 - bx0) * (by1 - by0)
                s = _vbcast(gs, l)[0]

                nk = jnp.where(act, (kc + (L - 1)) >> 4, 0)

                def chunk(j, miou):
                    base = j * L
                    vx0 = kx0[pl.ds(base, L)]
                    vy0 = ky0[pl.ds(base, L)]
                    vx1 = kx1[pl.ds(base, L)]
                    vy1 = ky1[pl.ds(base, L)]
                    va = karea[pl.ds(base, L)]
                    w = jnp.maximum(
                        jnp.minimum(bx1, vx1) - jnp.maximum(bx0, vx0), 0.0)
                    h = jnp.maximum(
                        jnp.minimum(by1, vy1) - jnp.maximum(by0, vy0), 0.0)
                    inter = w * h
                    union = jnp.maximum(barea + va - inter, 1e-9)
                    return jnp.maximum(miou, inter / union)

                miou = lax.fori_loop(0, nk, chunk,
                                     jnp.full((L,), -1.0, jnp.float32))
                sup = jnp.any(miou > NMS_THRESH)
                keep = jnp.logical_and(jnp.logical_not(sup), act)
                emit = jnp.logical_and(keep, s > SCORE_THRESH)

                @pl.when(keep)
                def _():
                    @pl.when(jnp.bitwise_and(kc, L - 1) == 0)
                    def _():
                        kx0[pl.ds(kc, L)] = zv
                        ky0[pl.ds(kc, L)] = zv
                        kx1[pl.ds(kc, L)] = zv
                        ky1[pl.ds(kc, L)] = zv
                        karea[pl.ds(kc, L)] = zv

                    kcv = [jnp.full((L,), kc, jnp.int32)]
                    plsc.store_scatter(kx0, kcv, bx0, mask=lane0)
                    plsc.store_scatter(ky0, kcv, by0, mask=lane0)
                    plsc.store_scatter(kx1, kcv, bx1, mask=lane0)
                    plsc.store_scatter(ky1, kcv, by1, mask=lane0)
                    plsc.store_scatter(karea, kcv, barea, mask=lane0)
                    kcnt_s[0] = kc + 1

                @pl.when(emit)
                def _():
                    bvv = jnp.where(lane == 0, bx0,
                                    jnp.where(lane == 1, by0,
                                              jnp.where(lane == 2, bx1, by1)))
                    plsc.store_scatter(outb_v, [4 * co + lane], bvv,
                                       mask=lane < 4)
                    plsc.store_scatter(outs_v,
                                       [jnp.full((L,), co, jnp.int32)],
                                       jnp.full((L,), s, jnp.float32),
                                       mask=lane0)
                    ocnt_s[0] = co + 1

            return bi

        def block(bi, carry):
            @pl.when(ocnt_s[0] < DETS_PER_IMG)
            def _():
                lax.fori_loop(0, B_PER_BLK, batch, bi)

            return carry

        lax.fori_loop(0, NBLK, block, jnp.int32(0))

        pltpu.sync_copy(outb_v, outb_hbm)
        pltpu.sync_copy(outs_v, outs_hbm)


@jax.jit
def kernel(boxes, scores):
    order = jnp.argsort(-scores).astype(jnp.int32)
    mesh = plsc.VectorSubcoreMesh(core_axis_name="c", subcore_axis_name="s",
                                  num_cores=1, num_subcores=1)
    nms = pl.kernel(
        _nms_body,
        out_type=(
            jax.ShapeDtypeStruct((OUTB_PAD,), jnp.float32),
            jax.ShapeDtypeStruct((OUTS_PAD,), jnp.float32),
        ),
        mesh=mesh,
        compiler_params=pltpu.CompilerParams(needs_layout_passes=False),
        scratch_types=[
            pltpu.VMEM((BOXPAD,), jnp.float32),
            pltpu.VMEM((SPAD,), jnp.float32),
            pltpu.VMEM((SPAD,), jnp.int32),
            pltpu.VMEM((KMAX,), jnp.float32),
            pltpu.VMEM((KMAX,), jnp.float32),
            pltpu.VMEM((KMAX,), jnp.float32),
            pltpu.VMEM((KMAX,), jnp.float32),
            pltpu.VMEM((KMAX,), jnp.float32),
            pltpu.VMEM((OUTB_PAD,), jnp.float32),
            pltpu.VMEM((OUTS_PAD,), jnp.float32),
            pltpu.SMEM((1,), jnp.int32),
            pltpu.SMEM((1,), jnp.int32),
            pltpu.SemaphoreType.DMA,
            pltpu.SemaphoreType.DMA,
            pltpu.SemaphoreType.DMA,
        ],
    )
    outb_flat, outs = nms(boxes.reshape(-1), scores, order)
    out_boxes = outb_flat[: 4 * DETS_PER_IMG].reshape(DETS_PER_IMG, 4)
    out_scores = outs[:DETS_PER_IMG]
    return (out_boxes, out_scores)


# speedup vs baseline: 1.0324x; 1.0324x over previous
import jax
import jax.numpy as jnp
from jax import lax
from jax.experimental import pallas as pl
from jax.experimental.pallas import tpu as pltpu
from jax.experimental.pallas import tpu_sc as plsc

N = 5000
NMS_THRESH = 0.5
SCORE_THRESH = 0.05
DETS_PER_IMG = 100

L = 16
KMAX = 5008
BOXPAD = 4 * N + L
SPAD = N + L
OUTB_PAD = 512
OUTS_PAD = 128


CB = 50
NBLK = N // CB

_GDN = lax.GatherDimensionNumbers(offset_dims=(), collapsed_slice_dims=(0,),
                                  start_index_map=(0,))


def _vbcast(v, k):
    idx = jnp.full((L, 1), k, jnp.int32)
    return lax.gather(v, idx, _GDN, slice_sizes=(1,),
                      mode=lax.GatherScatterMode.PROMISE_IN_BOUNDS)


def _nms_body(boxes_hbm, scores_hbm, order_hbm, outb_hbm, outs_hbm,
              boxes_v, scores_v, order_v,
              kx0, ky0, kx1, ky1, karea,
              outb_v, outs_v, kcnt_s, ocnt_s, sem_b, sem_s, sem_o):
    cid = lax.axis_index("c")
    sid = lax.axis_index("s")

    @pl.when(jnp.logical_and(cid == 0, sid == 0))
    def _():
        cp_b = pltpu.async_copy(boxes_hbm, boxes_v.at[pl.ds(0, 4 * N)], sem_b)
        cp_s = pltpu.async_copy(scores_hbm, scores_v.at[pl.ds(0, N)], sem_s)
        cp_o = pltpu.async_copy(order_hbm, order_v.at[pl.ds(0, N)], sem_o)

        zv = jnp.zeros((L,), jnp.float32)
        lane = lax.iota(jnp.int32, L)
        lane0 = lane == 0
        for j in range(OUTB_PAD // L):
            outb_v[pl.ds(j * L, L)] = zv
        for j in range(OUTS_PAD // L):
            outs_v[pl.ds(j * L, L)] = zv
        kcnt_s[0] = jnp.int32(0)
        ocnt_s[0] = jnp.int32(0)
        cp_b.wait()
        cp_s.wait()
        cp_o.wait()

        def candidate(ci, bi):
            i = bi * CB + ci
            kc = kcnt_s[0]
            co = ocnt_s[0]
            active = co < DETS_PER_IMG

            oi = order_v[pl.ds(i, L)][0]
            bv = boxes_v[pl.ds(4 * oi, L)]
            s = scores_v[pl.ds(oi, L)][0]

            bx0 = _vbcast(bv, 0)
            by0 = _vbcast(bv, 1)
            bx1 = _vbcast(bv, 2)
            by1 = _vbcast(bv, 3)
            barea = (bx1 - bx0) * (by1 - by0)

            nk = jnp.where(active, (kc + (L - 1)) >> 4, 0)

            def chunk(j, miou):
                base = j * L
                vx0 = kx0[pl.ds(base, L)]
                vy0 = ky0[pl.ds(base, L)]
                vx1 = kx1[pl.ds(base, L)]
                vy1 = ky1[pl.ds(base, L)]
                va = karea[pl.ds(base, L)]
                w = jnp.maximum(jnp.minimum(bx1, vx1) - jnp.maximum(bx0, vx0),
                                0.0)
                h = jnp.maximum(jnp.minimum(by1, vy1) - jnp.maximum(by0, vy0),
                                0.0)
                inter = w * h
                union = jnp.maximum(barea + va - inter, 1e-9)
                return jnp.maximum(miou, inter / union)

            miou = lax.fori_loop(0, nk, chunk,
                                 jnp.full((L,), -1.0, jnp.float32))
            sup = jnp.any(miou > NMS_THRESH)
            keep = jnp.logical_and(jnp.logical_not(sup), active)
            emit = jnp.logical_and(keep, s > SCORE_THRESH)

            @pl.when(keep)
            def _():
                @pl.when(jnp.bitwise_and(kc, L - 1) == 0)
                def _():
                    kx0[pl.ds(kc, L)] = zv
                    ky0[pl.ds(kc, L)] = zv
                    kx1[pl.ds(kc, L)] = zv
                    ky1[pl.ds(kc, L)] = zv
                    karea[pl.ds(kc, L)] = zv

                kcv = [jnp.full((L,), kc, jnp.int32)]
                plsc.store_scatter(kx0, kcv, bx0, mask=lane0)
                plsc.store_scatter(ky0, kcv, by0, mask=lane0)
                plsc.store_scatter(kx1, kcv, bx1, mask=lane0)
                plsc.store_scatter(ky1, kcv, by1, mask=lane0)
                plsc.store_scatter(karea, kcv, barea, mask=lane0)
                kcnt_s[0] = kc + 1

            @pl.when(emit)
            def _():
                plsc.store_scatter(outb_v, [4 * co + lane], bv,
                                   mask=lane < 4)
                plsc.store_scatter(outs_v, [jnp.full((L,), co, jnp.int32)],
                                   jnp.full((L,), s, jnp.float32),
                                   mask=lane0)
                ocnt_s[0] = co + 1

            return bi

        def block(bi, carry):
            @pl.when(ocnt_s[0] < DETS_PER_IMG)
            def _():
                lax.fori_loop(0, CB, candidate, bi)

            return carry

        lax.fori_loop(0, NBLK, block, jnp.int32(0))

        pltpu.sync_copy(outb_v, outb_hbm)
        pltpu.sync_copy(outs_v, outs_hbm)


@jax.jit
def kernel(boxes, scores):
    order = jnp.argsort(-scores).astype(jnp.int32)
    mesh = plsc.VectorSubcoreMesh(core_axis_name="c", subcore_axis_name="s",
                                  num_cores=1, num_subcores=1)
    nms = pl.kernel(
        _nms_body,
        out_type=(
            jax.ShapeDtypeStruct((OUTB_PAD,), jnp.float32),
            jax.ShapeDtypeStruct((OUTS_PAD,), jnp.float32),
        ),
        mesh=mesh,
        compiler_params=pltpu.CompilerParams(needs_layout_passes=False),
        scratch_types=[
            pltpu.VMEM((BOXPAD,), jnp.float32),
            pltpu.VMEM((SPAD,), jnp.float32),
            pltpu.VMEM((SPAD,), jnp.int32),
            pltpu.VMEM((KMAX,), jnp.float32),
            pltpu.VMEM((KMAX,), jnp.float32),
            pltpu.VMEM((KMAX,), jnp.float32),
            pltpu.VMEM((KMAX,), jnp.float32),
            pltpu.VMEM((KMAX,), jnp.float32),
            pltpu.VMEM((OUTB_PAD,), jnp.float32),
            pltpu.VMEM((OUTS_PAD,), jnp.float32),
            pltpu.SMEM((1,), jnp.int32),
            pltpu.SMEM((1,), jnp.int32),
            pltpu.SemaphoreType.DMA,
            pltpu.SemaphoreType.DMA,
            pltpu.SemaphoreType.DMA,
        ],
    )
    outb_flat, outs = nms(boxes.reshape(-1), scores, order)
    out_boxes = outb_flat[: 4 * DETS_PER_IMG].reshape(DETS_PER_IMG, 4)
    out_scores = outs[:DETS_PER_IMG]
    return (out_boxes, out_scores)
